# baseline (speedup 1.0000x reference)
import functools

import jax
import jax.numpy as jnp
from jax import lax
from jax.experimental import pallas as pl
from jax.experimental.pallas import tpu as pltpu
from jax.experimental.pallas import tpu_sc as plsc

N = 10000
C = 128
K = 16
HID = 512
NPAD = 10240
BIGSQ = 1e30
IMAX = 0x7FFFFFFF
IMIN = -0x80000000


def _ln_f(x, s, b):
    mu = jnp.mean(x, axis=-1, keepdims=True)
    var = jnp.mean((x - mu) ** 2, axis=-1, keepdims=True)
    return (x - mu) / jnp.sqrt(var + 1e-5) * s + b


RT_PRE = 1024


def _pre0_body(inp_ref, sW_ref, sb_ref, W_ref, b_ref, s_ref, be_ref,
               x_ref, h_ref, ht_ref, sqt_ref):
    i = pl.program_id(0)
    xb = jnp.maximum(jnp.dot(inp_ref[...], sW_ref[...],
                             preferred_element_type=jnp.float32)
                     + sb_ref[...], 0.0)
    x_ref[...] = xb
    hb = _ln_f(jnp.dot(xb, W_ref[...], preferred_element_type=jnp.float32)
               + b_ref[...], s_ref[...], be_ref[...])
    h_ref[...] = hb
    ht_ref[...] = hb.T.astype(jnp.bfloat16)
    sq = jnp.sum(hb * hb, axis=1)
    gcol = i * RT_PRE + lax.broadcasted_iota(jnp.int32, (1, RT_PRE), 1)
    sqt_ref[...] = jnp.where(gcol < N, sq.reshape(1, RT_PRE), BIGSQ)


def _pre1_body(x_ref, W_ref, b_ref, s_ref, be_ref,
               h_ref, ht_ref, sqt_ref):
    i = pl.program_id(0)
    hb = _ln_f(jnp.dot(x_ref[...], W_ref[...],
                       preferred_element_type=jnp.float32)
               + b_ref[...], s_ref[...], be_ref[...])
    h_ref[...] = hb
    ht_ref[...] = hb.T.astype(jnp.bfloat16)
    sq = jnp.sum(hb * hb, axis=1)
    gcol = i * RT_PRE + lax.broadcasted_iota(jnp.int32, (1, RT_PRE), 1)
    sqt_ref[...] = jnp.where(gcol < N, sq.reshape(1, RT_PRE), BIGSQ)


def _pre0(inp_p, stem_W, stem_b, W, b, s, be):
    g = NPAD // RT_PRE
    row = pl.BlockSpec((RT_PRE, C), lambda i: (i, 0))
    full = lambda shp: pl.BlockSpec(shp, lambda i: (0, 0))
    return pl.pallas_call(
        _pre0_body,
        grid=(g,),
        in_specs=[row, full((C, C)), full((1, C)), full((C, C)),
                  full((1, C)), full((1, C)), full((1, C))],
        out_specs=[row, row,
                   pl.BlockSpec((C, RT_PRE), lambda i: (0, i)),
                   pl.BlockSpec((1, RT_PRE), lambda i: (0, i))],
        out_shape=[jax.ShapeDtypeStruct((NPAD, C), jnp.float32),
                   jax.ShapeDtypeStruct((NPAD, C), jnp.float32),
                   jax.ShapeDtypeStruct((C, NPAD), jnp.bfloat16),
                   jax.ShapeDtypeStruct((1, NPAD), jnp.float32)],
    )(inp_p, stem_W, stem_b.reshape(1, C), W, b.reshape(1, C),
      s.reshape(1, C), be.reshape(1, C))


def _pre1(x, W, b, s, be):
    g = NPAD // RT_PRE
    row = pl.BlockSpec((RT_PRE, C), lambda i: (i, 0))
    full = lambda shp: pl.BlockSpec(shp, lambda i: (0, 0))
    return pl.pallas_call(
        _pre1_body,
        grid=(g,),
        in_specs=[row, full((C, C)), full((1, C)), full((1, C)),
                  full((1, C))],
        out_specs=[row,
                   pl.BlockSpec((C, RT_PRE), lambda i: (0, i)),
                   pl.BlockSpec((1, RT_PRE), lambda i: (0, i))],
        out_shape=[jax.ShapeDtypeStruct((NPAD, C), jnp.float32),
                   jax.ShapeDtypeStruct((C, NPAD), jnp.bfloat16),
                   jax.ShapeDtypeStruct((1, NPAD), jnp.float32)],
    )(x, W, b.reshape(1, C), s.reshape(1, C), be.reshape(1, C))


RT_KNN = 512
CCH = 5120
NCH = NPAD // CCH


NACC = 4
WACC = 256


def _pack_chunk(hq2, ht_ref, sqt_ref, c):
    cs = pl.ds(c * CCH, CCH)
    dot = jnp.dot(hq2, ht_ref[:, cs], preferred_element_type=jnp.float32)
    d = sqt_ref[0, cs][None, :] + dot
    bbits = lax.bitcast_convert_type(d, jnp.int32)
    mono = bbits ^ ((bbits >> 31) & jnp.int32(0x7FFFFFFF))
    ci = (lax.broadcasted_iota(jnp.int32, (RT_KNN, CCH), 1)
          + jnp.int32(c * CCH))
    return (mono & jnp.int32(-16384)) | ci


def _extract16(read_chunk, nchunks):
    prev = jnp.full((RT_KNN,), IMIN, jnp.int32)
    cols = []
    for _ in range(K):
        u = None
        for c in range(nchunks):
            v = read_chunk(c)
            flt = jnp.where(v > prev[:, None], v, IMAX)
            u = flt if u is None else jnp.minimum(u, flt)
        m = jnp.min(u, axis=1)
        prev = m
        cols.append(m)
    return cols


def _knn_body(hq_ref, ht_ref, sqt_ref, idx_ref):
    hq2 = (hq_ref[...] * -2.0).astype(jnp.bfloat16)
    accs = [jnp.full((RT_KNN, WACC), IMAX, jnp.int32) for _ in range(NACC)]
    ejmin = jnp.full((RT_KNN, WACC), IMAX, jnp.int32)
    for c in range(NCH):
        packed = _pack_chunk(hq2, ht_ref, sqt_ref, c)
        for f in range(CCH // WACC):
            v = packed[:, f * WACC:(f + 1) * WACC]
            for a in range(NACC):
                lo = jnp.minimum(accs[a], v)
                v = jnp.maximum(accs[a], v)
                accs[a] = lo
            ejmin = jnp.minimum(ejmin, v)

    cand = _extract16(lambda a: accs[a], NACC)
    m16 = cand[K - 1]
    ok = jnp.all(jnp.min(ejmin, axis=1) > m16)
    idx_ref[...] = jnp.concatenate(
        [(m & jnp.int32(16383)).reshape(RT_KNN, 1) for m in cand], axis=1)

    @pl.when(jnp.logical_not(ok))
    def _():
        full = _extract16(
            lambda c: _pack_chunk(hq2, ht_ref, sqt_ref, c), NCH)
        idx_ref[...] = jnp.concatenate(
            [(m & jnp.int32(16383)).reshape(RT_KNN, 1) for m in full],
            axis=1)


def _knn(h, ht, sqt, row0, nrows):
    g = nrows // RT_KNN
    r0 = row0 // RT_KNN
    return pl.pallas_call(
        _knn_body,
        grid=(g,),
        in_specs=[pl.BlockSpec((RT_KNN, C), lambda i: (i + r0, 0)),
                  pl.BlockSpec((C, NPAD), lambda i: (0, 0)),
                  pl.BlockSpec((1, NPAD), lambda i: (0, 0))],
        out_specs=pl.BlockSpec((RT_KNN, K), lambda i: (i, 0)),
        out_shape=jax.ShapeDtypeStruct((nrows, K), jnp.int32),
    )(h, ht, sqt)


NW = 32
CH_SC = 8


def _gmax_sc(per_w, h_hbm, idxf_hbm, out_hbm, idx_v, rows_v, out_v, sem):
    wid = lax.axis_index("s") * 2 + lax.axis_index("c")
    base = wid * per_w

    def chunk(ci, _):
        node0 = base + ci * CH_SC
        pltpu.sync_copy(idxf_hbm.at[pl.ds(node0 * K, CH_SC * K)], idx_v)
        pltpu.async_copy(h_hbm.at[idx_v], rows_v, sem).wait()

        def node(n, _):
            for v in range(C // 16):
                fs = pl.ds(v * 16, 16)
                acc = rows_v[n * K, fs]
                for kk in range(1, K):
                    acc = jnp.maximum(acc, rows_v[n * K + kk, fs])
                out_v[n, fs] = acc
            return 0

        lax.fori_loop(0, CH_SC, node, 0)
        pltpu.sync_copy(out_v, out_hbm.at[pl.ds(node0, CH_SC)])
        return 0

    lax.fori_loop(0, per_w // CH_SC, chunk, 0)


def _gather_max(h, idxf, nrows):
    mesh = plsc.VectorSubcoreMesh(core_axis_name="c", subcore_axis_name="s")
    f = functools.partial(
        pl.kernel,
        out_type=jax.ShapeDtypeStruct((nrows, C), jnp.float32),
        mesh=mesh,
        scratch_types=[
            pltpu.VMEM((CH_SC * K,), jnp.int32),
            pltpu.VMEM((CH_SC * K, C), jnp.float32),
            pltpu.VMEM((CH_SC, C), jnp.float32),
            pltpu.SemaphoreType.DMA,
        ],
    )(functools.partial(_gmax_sc, nrows // NW))
    return f(h, idxf)


RT_POST = 1024


def _post_body(h_ref, mxh_ref, x_ref, Wa_ref, Wb_ref, cb_ref,
               fW_ref, fb_ref, fs_ref, fbe_ref,
               f1W_ref, f1b_ref, f1s_ref, f1be_ref,
               f2W_ref, f2b_ref, f2s_ref, f2be_ref, out_ref):
    h = h_ref[...]
    mx = mxh_ref[...] - h
    t = jnp.maximum(
        jnp.dot(h, Wa_ref[...], preferred_element_type=jnp.float32)
        + jnp.dot(mx, Wb_ref[...], preferred_element_type=jnp.float32)
        + cb_ref[...], 0.0)
    t = _ln_f(jnp.dot(t, fW_ref[...], preferred_element_type=jnp.float32)
              + fb_ref[...], fs_ref[...], fbe_ref[...])
    x1 = t + x_ref[...]
    u = jnp.maximum(
        _ln_f(jnp.dot(x1, f1W_ref[...], preferred_element_type=jnp.float32)
              + f1b_ref[...], f1s_ref[...], f1be_ref[...]), 0.0)
    u = _ln_f(jnp.dot(u, f2W_ref[...], preferred_element_type=jnp.float32)
              + f2b_ref[...], f2s_ref[...], f2be_ref[...])
    out_ref[...] = u + x1


def _post(h, mxh, x, Wa, Wb, cb, fW, fb, fs, fbe,
          f1W, f1b, f1s, f1be, f2W, f2b, f2s, f2be):
    g = NPAD // RT_POST
    row = pl.BlockSpec((RT_POST, C), lambda i: (i, 0))
    full = lambda shp: pl.BlockSpec(shp, lambda i: (0,) * len(shp))
    r1 = lambda n: pl.BlockSpec((1, n), lambda i: (0, 0))
    return pl.pallas_call(
        _post_body,
        grid=(g,),
        in_specs=[row, row, row,
                  full((C, C)), full((C, C)), r1(C),
                  full((C, C)), r1(C), r1(C), r1(C),
                  full((C, HID)), r1(HID), r1(HID), r1(HID),
                  full((HID, C)), r1(C), r1(C), r1(C)],
        out_specs=row,
        out_shape=jax.ShapeDtypeStruct((NPAD, C), jnp.float32),
    )(h, mxh, x, Wa, Wb, cb.reshape(1, C),
      fW, fb.reshape(1, C), fs.reshape(1, C), fbe.reshape(1, C),
      f1W, f1b.reshape(1, HID), f1s.reshape(1, HID), f1be.reshape(1, HID),
      f2W, f2b.reshape(1, C), f2s.reshape(1, C), f2be.reshape(1, C))


RT_POOL = 1024


def _pool_body(x_ref, s_ref, b_ref, pW_ref, pb_ref, out_ref, acc_ref):
    i = pl.program_id(0)
    grow = i * RT_POOL + lax.broadcasted_iota(jnp.int32, (RT_POOL, 1), 0)
    xm = jnp.where(grow < N, x_ref[...], 0.0)
    part = jnp.sum(xm, axis=0).reshape(1, C)

    @pl.when(i == 0)
    def _():
        acc_ref[...] = jnp.zeros_like(acc_ref)

    acc_ref[...] += part

    @pl.when(i == pl.num_programs(0) - 1)
    def _():
        gm = acc_ref[...] * (1.0 / N)
        gm = _ln_f(gm, s_ref[...], b_ref[...])
        out_ref[...] = (jnp.dot(gm, pW_ref[...],
                                preferred_element_type=jnp.float32)
                        + pb_ref[...])


def _pool(x, s, b, pW, pb):
    g = NPAD // RT_POOL
    full = lambda shp: pl.BlockSpec(shp, lambda i: (0,) * len(shp))
    return pl.pallas_call(
        _pool_body,
        grid=(g,),
        in_specs=[pl.BlockSpec((RT_POOL, C), lambda i: (i, 0)),
                  full((1, C)), full((1, C)), full((C, C)), full((1, C))],
        out_specs=full((1, C)),
        out_shape=jax.ShapeDtypeStruct((1, C), jnp.float32),
        scratch_shapes=[pltpu.VMEM((1, C), jnp.float32)],
    )(x, s.reshape(1, C), b.reshape(1, C), pW, pb.reshape(1, C))


@jax.jit
def kernel(inputs, stem_W, stem_b, fc1_W, fc1_b, fc1_ln_s, fc1_ln_b,
           conv_W, conv_b, fc2_W, fc2_b, fc2_ln_s, fc2_ln_b,
           ffn1_W, ffn1_b, ffn1_ln_s, ffn1_ln_b,
           ffn2_W, ffn2_b, ffn2_ln_s, ffn2_ln_b,
           out_ln_s, out_ln_b, pred_W, pred_b):
    inp_p = jnp.pad(inputs[0], ((0, NPAD - N), (0, 0)))
    x = None
    for i in range(2):
        if i == 0:
            x, h, ht, sqt = _pre0(inp_p, stem_W, stem_b,
                                  fc1_W[0], fc1_b[0], fc1_ln_s[0],
                                  fc1_ln_b[0])
        else:
            h, ht, sqt = _pre1(x, fc1_W[i], fc1_b[i], fc1_ln_s[i],
                               fc1_ln_b[i])
        half = NPAD // 2
        idx_a = _knn(h, ht, sqt, 0, half)
        idx_b = _knn(h, ht, sqt, half, half)
        mxh_a = _gather_max(h, idx_a.reshape(half * K), half)
        mxh_b = _gather_max(h, idx_b.reshape(half * K), half)
        mxh = jnp.concatenate([mxh_a, mxh_b], axis=0)
        x = _post(h, mxh, x, conv_W[i][:C], conv_W[i][C:], conv_b[i],
                  fc2_W[i], fc2_b[i], fc2_ln_s[i], fc2_ln_b[i],
                  ffn1_W[i], ffn1_b[i], ffn1_ln_s[i], ffn1_ln_b[i],
                  ffn2_W[i], ffn2_b[i], ffn2_ln_s[i], ffn2_ln_b[i])
    return _pool(x, out_ln_s, out_ln_b, pred_W, pred_b)


# speedup vs baseline: 1.0007x; 1.0007x over previous
import functools

import jax
import jax.numpy as jnp
from jax import lax
from jax.experimental import pallas as pl
from jax.experimental.pallas import tpu as pltpu
from jax.experimental.pallas import tpu_sc as plsc

N = 10000
C = 128
K = 16
HID = 512
NPAD = 10240
BIGSQ = 1e30
IMAX = 0x7FFFFFFF
IMIN = -0x80000000


def _ln_f(x, s, b):
    mu = jnp.mean(x, axis=-1, keepdims=True)
    var = jnp.mean((x - mu) ** 2, axis=-1, keepdims=True)
    return (x - mu) / jnp.sqrt(var + 1e-5) * s + b


RT_PRE = 1024


def _pre0_body(inp_ref, sW_ref, sb_ref, W_ref, b_ref, s_ref, be_ref,
               x_ref, h_ref, ht_ref, sqt_ref):
    i = pl.program_id(0)
    xb = jnp.maximum(jnp.dot(inp_ref[...], sW_ref[...],
                             preferred_element_type=jnp.float32)
                     + sb_ref[...], 0.0)
    x_ref[...] = xb
    hb = _ln_f(jnp.dot(xb, W_ref[...], preferred_element_type=jnp.float32)
               + b_ref[...], s_ref[...], be_ref[...])
    h_ref[...] = hb
    ht_ref[...] = hb.T.astype(jnp.bfloat16)
    sq = jnp.sum(hb * hb, axis=1)
    gcol = i * RT_PRE + lax.broadcasted_iota(jnp.int32, (1, RT_PRE), 1)
    sqt_ref[...] = jnp.where(gcol < N, sq.reshape(1, RT_PRE), BIGSQ)


def _pre1_body(x_ref, W_ref, b_ref, s_ref, be_ref,
               h_ref, ht_ref, sqt_ref):
    i = pl.program_id(0)
    hb = _ln_f(jnp.dot(x_ref[...], W_ref[...],
                       preferred_element_type=jnp.float32)
               + b_ref[...], s_ref[...], be_ref[...])
    h_ref[...] = hb
    ht_ref[...] = hb.T.astype(jnp.bfloat16)
    sq = jnp.sum(hb * hb, axis=1)
    gcol = i * RT_PRE + lax.broadcasted_iota(jnp.int32, (1, RT_PRE), 1)
    sqt_ref[...] = jnp.where(gcol < N, sq.reshape(1, RT_PRE), BIGSQ)


def _pre0(inp_p, stem_W, stem_b, W, b, s, be):
    g = NPAD // RT_PRE
    row = pl.BlockSpec((RT_PRE, C), lambda i: (i, 0))
    full = lambda shp: pl.BlockSpec(shp, lambda i: (0, 0))
    return pl.pallas_call(
        _pre0_body,
        grid=(g,),
        in_specs=[row, full((C, C)), full((1, C)), full((C, C)),
                  full((1, C)), full((1, C)), full((1, C))],
        out_specs=[row, row,
                   pl.BlockSpec((C, RT_PRE), lambda i: (0, i)),
                   pl.BlockSpec((1, RT_PRE), lambda i: (0, i))],
        out_shape=[jax.ShapeDtypeStruct((NPAD, C), jnp.float32),
                   jax.ShapeDtypeStruct((NPAD, C), jnp.float32),
                   jax.ShapeDtypeStruct((C, NPAD), jnp.bfloat16),
                   jax.ShapeDtypeStruct((1, NPAD), jnp.float32)],
    )(inp_p, stem_W, stem_b.reshape(1, C), W, b.reshape(1, C),
      s.reshape(1, C), be.reshape(1, C))


def _pre1(x, W, b, s, be):
    g = NPAD // RT_PRE
    row = pl.BlockSpec((RT_PRE, C), lambda i: (i, 0))
    full = lambda shp: pl.BlockSpec(shp, lambda i: (0, 0))
    return pl.pallas_call(
        _pre1_body,
        grid=(g,),
        in_specs=[row, full((C, C)), full((1, C)), full((1, C)),
                  full((1, C))],
        out_specs=[row,
                   pl.BlockSpec((C, RT_PRE), lambda i: (0, i)),
                   pl.BlockSpec((1, RT_PRE), lambda i: (0, i))],
        out_shape=[jax.ShapeDtypeStruct((NPAD, C), jnp.float32),
                   jax.ShapeDtypeStruct((C, NPAD), jnp.bfloat16),
                   jax.ShapeDtypeStruct((1, NPAD), jnp.float32)],
    )(x, W, b.reshape(1, C), s.reshape(1, C), be.reshape(1, C))


RT_KNN = 512
CCH = 2048
NCH = NPAD // CCH


NACC = 4
WACC = 256


def _pack_chunk(hq2, ht_ref, sqt_ref, c):
    cs = pl.ds(c * CCH, CCH)
    dot = jnp.dot(hq2, ht_ref[:, cs], preferred_element_type=jnp.float32)
    d = sqt_ref[0, cs][None, :] + dot
    bbits = lax.bitcast_convert_type(d, jnp.int32)
    mono = bbits ^ ((bbits >> 31) & jnp.int32(0x7FFFFFFF))
    ci = (lax.broadcasted_iota(jnp.int32, (RT_KNN, CCH), 1)
          + jnp.int32(c * CCH))
    return (mono & jnp.int32(-16384)) | ci


def _extract16(read_chunk, nchunks):
    prev = jnp.full((RT_KNN,), IMIN, jnp.int32)
    cols = []
    for _ in range(K):
        u = None
        for c in range(nchunks):
            v = read_chunk(c)
            flt = jnp.where(v > prev[:, None], v, IMAX)
            u = flt if u is None else jnp.minimum(u, flt)
        m = jnp.min(u, axis=1)
        prev = m
        cols.append(m)
    return cols


def _knn_body(hq_ref, ht_ref, sqt_ref, idx_ref):
    hq2 = (hq_ref[...] * -2.0).astype(jnp.bfloat16)
    accs = [jnp.full((RT_KNN, WACC), IMAX, jnp.int32) for _ in range(NACC)]
    ejmin = jnp.full((RT_KNN, WACC), IMAX, jnp.int32)
    for c in range(NCH):
        packed = _pack_chunk(hq2, ht_ref, sqt_ref, c)
        for f in range(CCH // WACC):
            v = packed[:, f * WACC:(f + 1) * WACC]
            for a in range(NACC):
                lo = jnp.minimum(accs[a], v)
                v = jnp.maximum(accs[a], v)
                accs[a] = lo
            ejmin = jnp.minimum(ejmin, v)

    cand = _extract16(lambda a: accs[a], NACC)
    m16 = cand[K - 1]
    ok = jnp.all(jnp.min(ejmin, axis=1) > m16)
    idx_ref[...] = jnp.concatenate(
        [(m & jnp.int32(16383)).reshape(RT_KNN, 1) for m in cand], axis=1)

    @pl.when(jnp.logical_not(ok))
    def _():
        full = _extract16(
            lambda c: _pack_chunk(hq2, ht_ref, sqt_ref, c), NCH)
        idx_ref[...] = jnp.concatenate(
            [(m & jnp.int32(16383)).reshape(RT_KNN, 1) for m in full],
            axis=1)


def _knn(h, ht, sqt, row0, nrows):
    g = nrows // RT_KNN
    r0 = row0 // RT_KNN
    return pl.pallas_call(
        _knn_body,
        grid=(g,),
        in_specs=[pl.BlockSpec((RT_KNN, C), lambda i: (i + r0, 0)),
                  pl.BlockSpec((C, NPAD), lambda i: (0, 0)),
                  pl.BlockSpec((1, NPAD), lambda i: (0, 0))],
        out_specs=pl.BlockSpec((RT_KNN, K), lambda i: (i, 0)),
        out_shape=jax.ShapeDtypeStruct((nrows, K), jnp.int32),
    )(h, ht, sqt)


NW = 32
CH_SC = 8


def _gmax_sc(per_w, h_hbm, idxf_hbm, out_hbm, idx_v, rows_v, out_v, sem):
    wid = lax.axis_index("s") * 2 + lax.axis_index("c")
    base = wid * per_w

    def chunk(ci, _):
        node0 = base + ci * CH_SC
        pltpu.sync_copy(idxf_hbm.at[pl.ds(node0 * K, CH_SC * K)], idx_v)
        pltpu.async_copy(h_hbm.at[idx_v], rows_v, sem).wait()

        def node(n, _):
            for v in range(C // 16):
                fs = pl.ds(v * 16, 16)
                acc = rows_v[n * K, fs]
                for kk in range(1, K):
                    acc = jnp.maximum(acc, rows_v[n * K + kk, fs])
                out_v[n, fs] = acc
            return 0

        lax.fori_loop(0, CH_SC, node, 0)
        pltpu.sync_copy(out_v, out_hbm.at[pl.ds(node0, CH_SC)])
        return 0

    lax.fori_loop(0, per_w // CH_SC, chunk, 0)


def _gather_max(h, idxf, nrows):
    mesh = plsc.VectorSubcoreMesh(core_axis_name="c", subcore_axis_name="s")
    f = functools.partial(
        pl.kernel,
        out_type=jax.ShapeDtypeStruct((nrows, C), jnp.float32),
        mesh=mesh,
        scratch_types=[
            pltpu.VMEM((CH_SC * K,), jnp.int32),
            pltpu.VMEM((CH_SC * K, C), jnp.float32),
            pltpu.VMEM((CH_SC, C), jnp.float32),
            pltpu.SemaphoreType.DMA,
        ],
    )(functools.partial(_gmax_sc, nrows // NW))
    return f(h, idxf)


RT_POST = 1024


def _post_body(h_ref, mxh_ref, x_ref, Wa_ref, Wb_ref, cb_ref,
               fW_ref, fb_ref, fs_ref, fbe_ref,
               f1W_ref, f1b_ref, f1s_ref, f1be_ref,
               f2W_ref, f2b_ref, f2s_ref, f2be_ref, out_ref):
    h = h_ref[...]
    mx = mxh_ref[...] - h
    t = jnp.maximum(
        jnp.dot(h, Wa_ref[...], preferred_element_type=jnp.float32)
        + jnp.dot(mx, Wb_ref[...], preferred_element_type=jnp.float32)
        + cb_ref[...], 0.0)
    t = _ln_f(jnp.dot(t, fW_ref[...], preferred_element_type=jnp.float32)
              + fb_ref[...], fs_ref[...], fbe_ref[...])
    x1 = t + x_ref[...]
    u = jnp.maximum(
        _ln_f(jnp.dot(x1, f1W_ref[...], preferred_element_type=jnp.float32)
              + f1b_ref[...], f1s_ref[...], f1be_ref[...]), 0.0)
    u = _ln_f(jnp.dot(u, f2W_ref[...], preferred_element_type=jnp.float32)
              + f2b_ref[...], f2s_ref[...], f2be_ref[...])
    out_ref[...] = u + x1


def _post(h, mxh, x, Wa, Wb, cb, fW, fb, fs, fbe,
          f1W, f1b, f1s, f1be, f2W, f2b, f2s, f2be):
    g = NPAD // RT_POST
    row = pl.BlockSpec((RT_POST, C), lambda i: (i, 0))
    full = lambda shp: pl.BlockSpec(shp, lambda i: (0,) * len(shp))
    r1 = lambda n: pl.BlockSpec((1, n), lambda i: (0, 0))
    return pl.pallas_call(
        _post_body,
        grid=(g,),
        in_specs=[row, row, row,
                  full((C, C)), full((C, C)), r1(C),
                  full((C, C)), r1(C), r1(C), r1(C),
                  full((C, HID)), r1(HID), r1(HID), r1(HID),
                  full((HID, C)), r1(C), r1(C), r1(C)],
        out_specs=row,
        out_shape=jax.ShapeDtypeStruct((NPAD, C), jnp.float32),
    )(h, mxh, x, Wa, Wb, cb.reshape(1, C),
      fW, fb.reshape(1, C), fs.reshape(1, C), fbe.reshape(1, C),
      f1W, f1b.reshape(1, HID), f1s.reshape(1, HID), f1be.reshape(1, HID),
      f2W, f2b.reshape(1, C), f2s.reshape(1, C), f2be.reshape(1, C))


RT_POOL = 1024


def _pool_body(x_ref, s_ref, b_ref, pW_ref, pb_ref, out_ref, acc_ref):
    i = pl.program_id(0)
    grow = i * RT_POOL + lax.broadcasted_iota(jnp.int32, (RT_POOL, 1), 0)
    xm = jnp.where(grow < N, x_ref[...], 0.0)
    part = jnp.sum(xm, axis=0).reshape(1, C)

    @pl.when(i == 0)
    def _():
        acc_ref[...] = jnp.zeros_like(acc_ref)

    acc_ref[...] += part

    @pl.when(i == pl.num_programs(0) - 1)
    def _():
        gm = acc_ref[...] * (1.0 / N)
        gm = _ln_f(gm, s_ref[...], b_ref[...])
        out_ref[...] = (jnp.dot(gm, pW_ref[...],
                                preferred_element_type=jnp.float32)
                        + pb_ref[...])


def _pool(x, s, b, pW, pb):
    g = NPAD // RT_POOL
    full = lambda shp: pl.BlockSpec(shp, lambda i: (0,) * len(shp))
    return pl.pallas_call(
        _pool_body,
        grid=(g,),
        in_specs=[pl.BlockSpec((RT_POOL, C), lambda i: (i, 0)),
                  full((1, C)), full((1, C)), full((C, C)), full((1, C))],
        out_specs=full((1, C)),
        out_shape=jax.ShapeDtypeStruct((1, C), jnp.float32),
        scratch_shapes=[pltpu.VMEM((1, C), jnp.float32)],
    )(x, s.reshape(1, C), b.reshape(1, C), pW, pb.reshape(1, C))


@jax.jit
def kernel(inputs, stem_W, stem_b, fc1_W, fc1_b, fc1_ln_s, fc1_ln_b,
           conv_W, conv_b, fc2_W, fc2_b, fc2_ln_s, fc2_ln_b,
           ffn1_W, ffn1_b, ffn1_ln_s, ffn1_ln_b,
           ffn2_W, ffn2_b, ffn2_ln_s, ffn2_ln_b,
           out_ln_s, out_ln_b, pred_W, pred_b):
    inp_p = jnp.pad(inputs[0], ((0, NPAD - N), (0, 0)))
    x = None
    for i in range(2):
        if i == 0:
            x, h, ht, sqt = _pre0(inp_p, stem_W, stem_b,
                                  fc1_W[0], fc1_b[0], fc1_ln_s[0],
                                  fc1_ln_b[0])
        else:
            h, ht, sqt = _pre1(x, fc1_W[i], fc1_b[i], fc1_ln_s[i],
                               fc1_ln_b[i])
        half = NPAD // 2
        idx_a = _knn(h, ht, sqt, 0, half)
        idx_b = _knn(h, ht, sqt, half, half)
        mxh_a = _gather_max(h, idx_a.reshape(half * K), half)
        mxh_b = _gather_max(h, idx_b.reshape(half * K), half)
        mxh = jnp.concatenate([mxh_a, mxh_b], axis=0)
        x = _post(h, mxh, x, conv_W[i][:C], conv_W[i][C:], conv_b[i],
                  fc2_W[i], fc2_b[i], fc2_ln_s[i], fc2_ln_b[i],
                  ffn1_W[i], ffn1_b[i], ffn1_ln_s[i], ffn1_ln_b[i],
                  ffn2_W[i], ffn2_b[i], ffn2_ln_s[i], ffn2_ln_b[i])
    return _pool(x, out_ln_s, out_ln_b, pred_W, pred_b)
